# Initial kernel scaffold; baseline (speedup 1.0000x reference)
import jax
import jax.numpy as jnp
from jax.experimental import pallas as pl


def kernel(x, edge_index, W1l, W1r, b1, W2l, W2r, b2):
    raise NotImplementedError("write your pallas kernel here")


# speedup vs baseline: 10.4720x; 10.4720x over previous
import functools

import jax
import jax.numpy as jnp
from jax import lax
from jax.experimental import pallas as pl
from jax.experimental.pallas import tpu as pltpu
from jax.experimental.pallas import tpu_sc as plsc

N = 10000
E = 320000
D_IN = 128
D_HID = 8
N_CLASSES = 40

NC = 2
NS = 16
NW = NC * NS

CHUNK = 128
CPW = 80
E_PAD = NW * CPW * CHUNK
TW = 16
ACC_ROWS = 10112
DUMMY = N
ZROWS = ACC_ROWS // NS

ROWS_BLK = 1000
GRID = N // ROWS_BLK


@functools.cache
def _segment_accumulate_sc():
  mesh = plsc.VectorSubcoreMesh(
      core_axis_name="c", subcore_axis_name="s", num_cores=NC,
      num_subcores=NS)

  @functools.partial(
      pl.kernel,
      out_type=jax.ShapeDtypeStruct((NC, ACC_ROWS, TW), jnp.float32),
      mesh=mesh,
      scratch_types=[
          pltpu.VMEM((CPW, CHUNK), jnp.int32),
          pltpu.VMEM((CPW, CHUNK), jnp.int32),
          pltpu.VMEM((CHUNK, TW), jnp.float32),
          pltpu.VMEM_SHARED((ACC_ROWS, TW), jnp.float32),
      ],
      compiler_params=pltpu.CompilerParams(use_tc_tiling_on_sc=False),
  )
  def seg_acc(table_hbm, src_hbm, dst_hbm, zeros_hbm, out_hbm,
              sidx_v, didx_v, rows_v, acc_sh):
    c = lax.axis_index("c")
    s = lax.axis_index("s")
    wid = s * NC + c
    pltpu.sync_copy(zeros_hbm.at[pl.ds(s * ZROWS, ZROWS)],
                    acc_sh.at[pl.ds(s * ZROWS, ZROWS)])
    pltpu.sync_copy(src_hbm.at[wid], sidx_v)
    pltpu.sync_copy(dst_hbm.at[wid], didx_v)
    plsc.subcore_barrier()

    def body(j, carry):
      pltpu.sync_copy(table_hbm.at[sidx_v.at[j]], rows_v)
      pltpu.sync_copy(rows_v, acc_sh.at[didx_v.at[j]], add=True)
      return carry

    lax.fori_loop(0, CPW, body, 0)
    plsc.subcore_barrier()
    pltpu.sync_copy(acc_sh.at[pl.ds(s * ZROWS, ZROWS)],
                    out_hbm.at[c, pl.ds(s * ZROWS, ZROWS)])

  return seg_acc


def _l1_body(x_ref, w_ref, b_ref, t_ref, q_ref):
  h = jnp.dot(x_ref[...], w_ref[...],
              preferred_element_type=jnp.float32,
              precision=lax.Precision.HIGHEST)
  ones = jnp.ones((ROWS_BLK, 1), jnp.float32)
  zeros = jnp.zeros((ROWS_BLK, TW - D_HID - 1), jnp.float32)
  t_ref[...] = jnp.concatenate([h[:, 0:D_HID], ones, zeros], axis=1)
  q_ref[...] = h[:, D_HID:2 * D_HID] + b_ref[...]


def _l2_body(parts_ref, q_ref, w2r_ref, b2_ref, t2_ref, hr2_ref):
  ssum = parts_ref[0] + parts_ref[1]
  deg = ssum[:, D_HID:D_HID + 1]
  rdeg = 1.0 / jnp.maximum(deg, 1.0)
  h1 = jnp.maximum(ssum[:, 0:D_HID] * rdeg + q_ref[...], 0.0)
  zeros = jnp.zeros((ROWS_BLK, TW - D_HID - 1), jnp.float32)
  t2_ref[...] = jnp.concatenate([h1, rdeg, zeros], axis=1)
  hr2_ref[...] = jnp.dot(h1, w2r_ref[...],
                         preferred_element_type=jnp.float32,
                         precision=lax.Precision.HIGHEST) + b2_ref[...]


def _out_body(parts_ref, t2_ref, hr2_ref, w2l_ref, o_ref):
  ssum = parts_ref[0] + parts_ref[1]
  rdeg = t2_ref[:, D_HID:D_HID + 1]
  agg2 = ssum[:, 0:D_HID] * rdeg
  logits = jnp.dot(agg2, w2l_ref[...],
                   preferred_element_type=jnp.float32,
                   precision=lax.Precision.HIGHEST) + hr2_ref[...]
  m = jnp.max(logits, axis=1, keepdims=True)
  z = logits - m
  lse = jnp.log(jnp.sum(jnp.exp(z), axis=1, keepdims=True))
  o_ref[...] = z - lse


_row_spec = lambda w: pl.BlockSpec((ROWS_BLK, w), lambda i: (i, 0))
_full_spec = lambda r, w: pl.BlockSpec((r, w), lambda i: (0, 0))
_parts_spec = pl.BlockSpec((NC, ROWS_BLK, TW), lambda i: (0, i, 0))

_l1_call = pl.pallas_call(
    _l1_body,
    grid=(GRID,),
    in_specs=[_row_spec(D_IN), _full_spec(D_IN, TW), _full_spec(1, D_HID)],
    out_specs=[_row_spec(TW), _row_spec(D_HID)],
    out_shape=[jax.ShapeDtypeStruct((N, TW), jnp.float32),
               jax.ShapeDtypeStruct((N, D_HID), jnp.float32)],
)

_l2_call = pl.pallas_call(
    _l2_body,
    grid=(GRID,),
    in_specs=[_parts_spec, _row_spec(D_HID), _full_spec(D_HID, N_CLASSES),
              _full_spec(1, N_CLASSES)],
    out_specs=[_row_spec(TW), _row_spec(N_CLASSES)],
    out_shape=[jax.ShapeDtypeStruct((N, TW), jnp.float32),
               jax.ShapeDtypeStruct((N, N_CLASSES), jnp.float32)],
)

_out_call = pl.pallas_call(
    _out_body,
    grid=(GRID,),
    in_specs=[_parts_spec, _row_spec(TW), _row_spec(N_CLASSES),
              _full_spec(D_HID, N_CLASSES)],
    out_specs=_row_spec(N_CLASSES),
    out_shape=jax.ShapeDtypeStruct((N, N_CLASSES), jnp.float32),
)


def kernel(x, edge_index, W1l, W1r, b1, W2l, W2r, b2):
  src = edge_index[0].astype(jnp.int32)
  dst = edge_index[1].astype(jnp.int32)
  pad = E_PAD - E
  src_p = jnp.concatenate([src, jnp.zeros((pad,), jnp.int32)])
  dst_p = jnp.concatenate([dst, jnp.full((pad,), DUMMY, jnp.int32)])
  src_p = src_p.reshape(NW, CPW, CHUNK)
  dst_p = dst_p.reshape(NW, CPW, CHUNK)
  zeros_acc = jnp.zeros((ACC_ROWS, TW), jnp.float32)

  W1 = jnp.concatenate([W1l, W1r], axis=1)

  seg_acc = _segment_accumulate_sc()
  t1, q1b = _l1_call(x, W1, b1.reshape(1, D_HID))
  parts1 = seg_acc(t1, src_p, dst_p, zeros_acc)
  t2, hr2 = _l2_call(parts1, q1b, W2r, b2.reshape(1, N_CLASSES))
  parts2 = seg_acc(t2, src_p, dst_p, zeros_acc)
  return _out_call(parts2, t2, hr2, W2l)


# speedup vs baseline: 20.3999x; 1.9480x over previous
import functools

import jax
import jax.numpy as jnp
from jax import lax
from jax.experimental import pallas as pl
from jax.experimental.pallas import tpu as pltpu
from jax.experimental.pallas import tpu_sc as plsc

N = 10000
E = 320000
D_IN = 128
D_HID = 8
N_CLASSES = 40

NC = 2
NS = 16
NW = NC * NS

CHUNK = 128
CPW = 80
E_PAD = NW * CPW * CHUNK
TW = 16
ACC_ROWS = 10112
DUMMY = N
ZROWS = ACC_ROWS // NS

ROWS_BLK = ACC_ROWS // 16
GRID = 16
OUT_BLK = 1000
OUT_GRID = N // OUT_BLK


@functools.cache
def _segment_accumulate_sc():
  mesh = plsc.VectorSubcoreMesh(
      core_axis_name="c", subcore_axis_name="s", num_cores=NC,
      num_subcores=NS)

  @functools.partial(
      pl.kernel,
      out_type=jax.ShapeDtypeStruct((NC, ACC_ROWS, TW), jnp.float32),
      mesh=mesh,
      scratch_types=[
          pltpu.VMEM((CPW, CHUNK), jnp.int32),
          pltpu.VMEM((CPW, CHUNK), jnp.int32),
          pltpu.VMEM((CHUNK, TW), jnp.float32),
          pltpu.VMEM((CHUNK, TW), jnp.float32),
          pltpu.VMEM_SHARED((ACC_ROWS, TW), jnp.float32),
          pltpu.VMEM_SHARED((ACC_ROWS, TW), jnp.float32),
          pltpu.SemaphoreType.DMA,
          pltpu.SemaphoreType.DMA,
      ],
      compiler_params=pltpu.CompilerParams(use_tc_tiling_on_sc=False),
  )
  def seg_acc(table_hbm, src_hbm, dst_hbm, zeros_hbm, out_hbm,
              sidx_v, didx_v, rows0_v, rows1_v, tbl_sh, acc_sh,
              gsem0, gsem1):
    c = lax.axis_index("c")
    s = lax.axis_index("s")
    wid = s * NC + c
    pltpu.sync_copy(table_hbm.at[pl.ds(s * ZROWS, ZROWS)],
                    tbl_sh.at[pl.ds(s * ZROWS, ZROWS)])
    pltpu.sync_copy(zeros_hbm.at[pl.ds(s * ZROWS, ZROWS)],
                    acc_sh.at[pl.ds(s * ZROWS, ZROWS)])
    pltpu.sync_copy(src_hbm.at[wid], sidx_v)
    pltpu.sync_copy(dst_hbm.at[wid], didx_v)
    plsc.subcore_barrier()

    bufs = (rows0_v, rows1_v)
    sems = (gsem0, gsem1)
    pltpu.async_copy(tbl_sh.at[sidx_v.at[0]], rows0_v, gsem0)

    def round_fn(r, carry):
      for b in range(2):
        j = r * 2 + b
        jn = jnp.minimum(j + 1, CPW - 1)
        pltpu.async_copy(tbl_sh.at[sidx_v.at[jn]], bufs[1 - b], sems[1 - b])
        pltpu.make_async_copy(tbl_sh.at[sidx_v.at[0]], bufs[b],
                              sems[b]).wait()
        pltpu.sync_copy(bufs[b], acc_sh.at[didx_v.at[j]], add=True)
      return carry

    lax.fori_loop(0, CPW // 2, round_fn, 0)
    pltpu.make_async_copy(tbl_sh.at[sidx_v.at[0]], rows0_v, gsem0).wait()
    plsc.subcore_barrier()
    pltpu.sync_copy(acc_sh.at[pl.ds(s * ZROWS, ZROWS)],
                    out_hbm.at[c, pl.ds(s * ZROWS, ZROWS)])

  return seg_acc


def _l1_body(x_ref, w_ref, b_ref, t_ref, q_ref):
  h = jnp.dot(x_ref[...], w_ref[...],
              preferred_element_type=jnp.float32,
              precision=lax.Precision.HIGHEST)
  ones = jnp.ones((ROWS_BLK, 1), jnp.float32)
  zeros = jnp.zeros((ROWS_BLK, TW - D_HID - 1), jnp.float32)
  t_ref[...] = jnp.concatenate([h[:, 0:D_HID], ones, zeros], axis=1)
  q_ref[...] = h[:, D_HID:2 * D_HID] + b_ref[...]


def _l2_body(parts_ref, q_ref, w2r_ref, b2_ref, t2_ref, hr2_ref):
  ssum = parts_ref[0] + parts_ref[1]
  deg = ssum[:, D_HID:D_HID + 1]
  rdeg = 1.0 / jnp.maximum(deg, 1.0)
  h1 = jnp.maximum(ssum[:, 0:D_HID] * rdeg + q_ref[...], 0.0)
  zeros = jnp.zeros((ROWS_BLK, TW - D_HID - 1), jnp.float32)
  t2_ref[...] = jnp.concatenate([h1, rdeg, zeros], axis=1)
  hr2_ref[...] = jnp.dot(h1, w2r_ref[...],
                         preferred_element_type=jnp.float32,
                         precision=lax.Precision.HIGHEST) + b2_ref[...]


def _out_body(parts_ref, t2_ref, hr2_ref, w2l_ref, o_ref):
  ssum = parts_ref[0] + parts_ref[1]
  rdeg = t2_ref[:, D_HID:D_HID + 1]
  agg2 = ssum[:, 0:D_HID] * rdeg
  logits = jnp.dot(agg2, w2l_ref[...],
                   preferred_element_type=jnp.float32,
                   precision=lax.Precision.HIGHEST) + hr2_ref[...]
  m = jnp.max(logits, axis=1, keepdims=True)
  z = logits - m
  lse = jnp.log(jnp.sum(jnp.exp(z), axis=1, keepdims=True))
  o_ref[...] = z - lse


_row_spec = lambda blk, w: pl.BlockSpec((blk, w), lambda i: (i, 0))
_full_spec = lambda r, w: pl.BlockSpec((r, w), lambda i: (0, 0))
_parts_spec = lambda blk: pl.BlockSpec((NC, blk, TW), lambda i: (0, i, 0))

_l1_call = pl.pallas_call(
    _l1_body,
    grid=(GRID,),
    in_specs=[_row_spec(ROWS_BLK, D_IN), _full_spec(D_IN, TW),
              _full_spec(1, D_HID)],
    out_specs=[_row_spec(ROWS_BLK, TW), _row_spec(ROWS_BLK, D_HID)],
    out_shape=[jax.ShapeDtypeStruct((ACC_ROWS, TW), jnp.float32),
               jax.ShapeDtypeStruct((ACC_ROWS, D_HID), jnp.float32)],
)

_l2_call = pl.pallas_call(
    _l2_body,
    grid=(GRID,),
    in_specs=[_parts_spec(ROWS_BLK), _row_spec(ROWS_BLK, D_HID),
              _full_spec(D_HID, N_CLASSES), _full_spec(1, N_CLASSES)],
    out_specs=[_row_spec(ROWS_BLK, TW), _row_spec(ROWS_BLK, N_CLASSES)],
    out_shape=[jax.ShapeDtypeStruct((ACC_ROWS, TW), jnp.float32),
               jax.ShapeDtypeStruct((ACC_ROWS, N_CLASSES), jnp.float32)],
)

_out_call = pl.pallas_call(
    _out_body,
    grid=(OUT_GRID,),
    in_specs=[_parts_spec(OUT_BLK), _row_spec(OUT_BLK, TW),
              _row_spec(OUT_BLK, N_CLASSES), _full_spec(D_HID, N_CLASSES)],
    out_specs=_row_spec(OUT_BLK, N_CLASSES),
    out_shape=jax.ShapeDtypeStruct((N, N_CLASSES), jnp.float32),
)


def kernel(x, edge_index, W1l, W1r, b1, W2l, W2r, b2):
  src = edge_index[0].astype(jnp.int32)
  dst = edge_index[1].astype(jnp.int32)
  pad = E_PAD - E
  src_p = jnp.concatenate([src, jnp.zeros((pad,), jnp.int32)])
  dst_p = jnp.concatenate([dst, jnp.full((pad,), DUMMY, jnp.int32)])
  src_p = src_p.reshape(NW, CPW, CHUNK)
  dst_p = dst_p.reshape(NW, CPW, CHUNK)
  zeros_acc = jnp.zeros((ACC_ROWS, TW), jnp.float32)

  W1 = jnp.concatenate([W1l, W1r], axis=1)

  seg_acc = _segment_accumulate_sc()
  t1, q1b = _l1_call(x, W1, b1.reshape(1, D_HID))
  parts1 = seg_acc(t1, src_p, dst_p, zeros_acc)
  t2, hr2 = _l2_call(parts1, q1b, W2r, b2.reshape(1, N_CLASSES))
  parts2 = seg_acc(t2, src_p, dst_p, zeros_acc)
  return _out_call(parts2, t2, hr2, W2l)
